# baseline (speedup 1.0000x reference)
import jax
import jax.numpy as jnp
from jax import lax
from jax.experimental import pallas as pl

ROWS, COLS, DEPTH = 16384, 26, 257
BLK = 256


def _onehot_block(x_ref, o_ref):
    idx = x_ref[...]
    k = lax.broadcasted_iota(jnp.int32, (BLK, COLS, DEPTH), 2)
    o_ref[...] = (idx[:, :, None] == k).astype(jnp.float32)


def kernel(x):
    x = x.astype(jnp.int32)
    return pl.pallas_call(
        _onehot_block,
        grid=(ROWS // BLK,),
        in_specs=[pl.BlockSpec((BLK, COLS), lambda i: (i, 0))],
        out_specs=pl.BlockSpec((BLK, COLS, DEPTH), lambda i: (i, 0, 0)),
        out_shape=jax.ShapeDtypeStruct((ROWS, COLS, DEPTH), jnp.float32),
    )(x)


# speedup vs baseline: 3.7338x; 3.7338x over previous
import jax
import jax.numpy as jnp
from jax import lax
from jax.experimental import pallas as pl

ROWS, COLS, DEPTH = 16384, 26, 257
KB = 32
NKB = 256 // KB


def _main_body(x_ref, o_ref):
    xj = x_ref[0]
    k0 = pl.program_id(1) * KB
    k = lax.broadcasted_iota(jnp.int32, (KB, ROWS), 0) + k0
    o_ref[0] = (jnp.broadcast_to(xj, (KB, ROWS)) == k).astype(jnp.float32)


def _plane_body(x_ref, o_ref):
    o_ref[0] = (x_ref[0] == DEPTH - 1).astype(jnp.float32)


def kernel(x):
    xT = x.astype(jnp.int32).T.reshape(COLS, 1, ROWS)
    main = pl.pallas_call(
        _main_body,
        grid=(COLS, NKB),
        in_specs=[pl.BlockSpec((1, 1, ROWS), lambda j, kb: (j, 0, 0))],
        out_specs=pl.BlockSpec((1, KB, ROWS), lambda j, kb: (j, kb, 0)),
        out_shape=jax.ShapeDtypeStruct((COLS, DEPTH, ROWS), jnp.float32),
    )(xT)
    plane = pl.pallas_call(
        _plane_body,
        grid=(COLS,),
        in_specs=[pl.BlockSpec((1, 1, ROWS), lambda j: (j, 0, 0))],
        out_specs=pl.BlockSpec((1, 1, ROWS), lambda j: (j, 0, 0)),
        out_shape=jax.ShapeDtypeStruct((COLS, 1, ROWS), jnp.float32),
    )(xT)
    full = lax.dynamic_update_slice(main, plane, (0, DEPTH - 1, 0))
    return full.transpose(2, 0, 1)


# speedup vs baseline: 4.7924x; 1.2835x over previous
import jax
import jax.numpy as jnp
from jax import lax
from jax.experimental import pallas as pl

ROWS, COLS, DEPTH = 16384, 26, 257
KB = 128
NKB = 256 // KB


def _main_body(x_ref, o_ref):
    xj = x_ref[0]
    k0 = pl.program_id(1) * KB
    k = lax.broadcasted_iota(jnp.int32, (KB, ROWS), 0) + k0
    o_ref[0] = (jnp.broadcast_to(xj, (KB, ROWS)) == k).astype(jnp.float32)


def kernel(x):
    xT = x.astype(jnp.int32).T.reshape(COLS, 1, ROWS)
    main = pl.pallas_call(
        _main_body,
        grid=(COLS, NKB),
        in_specs=[pl.BlockSpec((1, 1, ROWS), lambda j, t: (j, 0, 0))],
        out_specs=pl.BlockSpec((1, KB, ROWS), lambda j, t: (j, t, 0)),
        out_shape=jax.ShapeDtypeStruct((COLS, DEPTH, ROWS), jnp.float32),
    )(xT)
    plane = (xT == DEPTH - 1).astype(jnp.float32)
    full = lax.dynamic_update_slice(main, plane, (0, DEPTH - 1, 0))
    return full.transpose(2, 0, 1)


# speedup vs baseline: 6.1203x; 1.2771x over previous
import jax
import jax.numpy as jnp
from jax import lax
from jax.experimental import pallas as pl

ROWS, COLS, DEPTH = 16384, 26, 257
KB = 88
NKB = -(-DEPTH // KB)


def _body(x_ref, o_ref):
    xj = x_ref[0]
    k = lax.broadcasted_iota(jnp.int32, (KB, ROWS), 0) + pl.program_id(1) * KB
    o_ref[0] = (jnp.broadcast_to(xj, (KB, ROWS)) == k).astype(jnp.float32)


def kernel(x):
    xT = x.astype(jnp.int32).T.reshape(COLS, 1, ROWS)
    t = pl.pallas_call(
        _body,
        grid=(COLS, NKB),
        in_specs=[pl.BlockSpec((1, 1, ROWS), lambda j, t: (j, 0, 0))],
        out_specs=pl.BlockSpec((1, KB, ROWS), lambda j, t: (j, t, 0)),
        out_shape=jax.ShapeDtypeStruct((COLS, DEPTH, ROWS), jnp.float32),
    )(xT)
    return t.transpose(2, 0, 1)
